# Initial kernel scaffold; baseline (speedup 1.0000x reference)
import jax
import jax.numpy as jnp
from jax.experimental import pallas as pl


def kernel(seq_len, table):
    raise NotImplementedError("write your pallas kernel here")


# speedup vs baseline: 2.8868x; 2.8868x over previous
import functools

import jax
import jax.numpy as jnp
from jax import lax
from jax.experimental import pallas as pl
from jax.experimental.pallas import tpu as pltpu
from jax.experimental.pallas import tpu_sc as plsc

_MAX_DIST = 128
_D = 256
_L = 512
_T_ROWS = 2 * _MAX_DIST + 1

_NC = 2
_NS = 16
_NW = _NC * _NS
_ROWS_PER_W = _L // _NW

_P = 64
_C_ROWS = 2 * _P + _T_ROWS
_W = 64
_NCHUNK = _L // _W
_C_OFF = _T_ROWS + _P - 1 + _MAX_DIST - (_L - 1)
_E_BASE = _L - 1 - _MAX_DIST - _P
_T256_FLANK = _P + _T_ROWS


def _body(table_hbm, out_hbm, c_ref, sem):
    wid = lax.axis_index("s") * _NC + lax.axis_index("c")

    pltpu.sync_copy(table_hbm, c_ref.at[pl.ds(_P, _T_ROWS)])

    nv = _D // 16
    v0 = [c_ref[_P, pl.ds(16 * k, 16)] for k in range(nv)]
    v1 = [c_ref[_P + _T_ROWS - 1, pl.ds(16 * k, 16)] for k in range(nv)]

    def _fill_row(r, _):
        for k in range(nv):
            c_ref[r, pl.ds(16 * k, 16)] = v0[k]
            c_ref[_T256_FLANK + r, pl.ds(16 * k, 16)] = v1[k]
        return 0

    lax.fori_loop(0, _P, _fill_row, 0)

    row0 = wid * _ROWS_PER_W
    for r in range(_ROWS_PER_W):
        i = row0 + r
        handles = []
        for c in range(_NCHUNK):
            k0 = (_L - 1) - i + _W * c
            src = jnp.where(
                k0 + _W <= _L - _MAX_DIST,
                0,
                jnp.where(k0 >= _L + _MAX_DIST - 1,
                          _T256_FLANK,
                          k0 - _E_BASE))
            handles.append(
                pltpu.async_copy(c_ref.at[pl.ds(src, _W)],
                                 out_hbm.at[i, pl.ds(c * _W, _W)],
                                 sem))
        for h in handles:
            h.wait()


@jax.jit
def _rpe(table):
    mesh = plsc.VectorSubcoreMesh(core_axis_name="c", subcore_axis_name="s")
    return pl.kernel(
        _body,
        out_type=jax.ShapeDtypeStruct((_L, _L, _D), jnp.float32),
        mesh=mesh,
        scratch_types=[
            pltpu.VMEM((_C_ROWS, _D), jnp.float32),
            pltpu.SemaphoreType.DMA,
        ],
        compiler_params=pltpu.CompilerParams(use_tc_tiling_on_sc=False),
    )(table)


def kernel(seq_len, table):
    return _rpe(table)


# speedup vs baseline: 2.9224x; 1.0123x over previous
import jax
import jax.numpy as jnp
from jax import lax
from jax.experimental import pallas as pl
from jax.experimental.pallas import tpu as pltpu
from jax.experimental.pallas import tpu_sc as plsc

_MAX_DIST = 128
_D = 256
_L = 512
_T_ROWS = 2 * _MAX_DIST + 1

_NC = 2
_NS = 16
_NW = _NC * _NS

_DH = _D // 2
_E_ROWS = 2 * (_L - 1) + 1
_FLANK = _L - 1 - _MAX_DIST
_NG = _NW // 2
_ROWS_PER_G = _L // _NG


def _body(table_hbm, out_hbm, e_ref, sem):
    wid = lax.axis_index("s") * _NC + lax.axis_index("c")
    h = wid % 2
    g = wid // 2

    pltpu.sync_copy(table_hbm.at[:, pl.ds(h * _DH, _DH)],
                    e_ref.at[pl.ds(_FLANK, _T_ROWS)])

    nv = _DH // 16
    v0 = [e_ref[_FLANK, pl.ds(16 * k, 16)] for k in range(nv)]
    v1 = [e_ref[_FLANK + _T_ROWS - 1, pl.ds(16 * k, 16)] for k in range(nv)]

    def _fill_row(r, _):
        for k in range(nv):
            e_ref[r, pl.ds(16 * k, 16)] = v0[k]
            e_ref[_FLANK + _T_ROWS + r, pl.ds(16 * k, 16)] = v1[k]
        return 0

    lax.fori_loop(0, _FLANK, _fill_row, 0)

    row0 = g * _ROWS_PER_G
    handles = []
    for r in range(_ROWS_PER_G):
        i = row0 + r
        handles.append(
            pltpu.async_copy(e_ref.at[pl.ds((_L - 1) - i, _L)],
                             out_hbm.at[i, :, pl.ds(h * _DH, _DH)],
                             sem))
    for hd in handles:
        hd.wait()


@jax.jit
def _rpe(table):
    mesh = plsc.VectorSubcoreMesh(core_axis_name="c", subcore_axis_name="s")
    return pl.kernel(
        _body,
        out_type=jax.ShapeDtypeStruct((_L, _L, _D), jnp.float32),
        mesh=mesh,
        scratch_types=[
            pltpu.VMEM((_E_ROWS, _DH), jnp.float32),
            pltpu.SemaphoreType.DMA,
        ],
        compiler_params=pltpu.CompilerParams(use_tc_tiling_on_sc=False),
    )(table)


def kernel(seq_len, table):
    return _rpe(table)


# speedup vs baseline: 7.6500x; 2.6178x over previous
import jax
import jax.numpy as jnp
from jax import lax
from jax.experimental import pallas as pl
from jax.experimental.pallas import tpu as pltpu
from jax.experimental.pallas import tpu_sc as plsc

_MAX_DIST = 128
_D = 256
_L = 512
_T_ROWS = 2 * _MAX_DIST + 1

_NC = 2
_NS = 16

_DH = _D // 2
_E_ROWS = 2 * (_L - 1) + 1
_FLANK = _L - 1 - _MAX_DIST
_E8_ROWS = 1032
_W_ROWS = 1016
_NCLS = 16
_ROWS_PER_CLS = _L // _NCLS


def _body(e8_hbm, out_hbm, w_ref, sem):
    wid = lax.axis_index("s") * _NC + lax.axis_index("c")
    h = wid % 2
    c16 = wid // 2
    s0 = (_NCLS - 1) - c16
    p = lax.rem(c16 + 1, 8)
    S = pl.multiple_of(s0 + p, 8)

    pltpu.sync_copy(e8_hbm.at[p, pl.ds(S, _W_ROWS), pl.ds(h * _DH, _DH)],
                    w_ref)

    handles = []
    for t in range(_ROWS_PER_CLS):
        i = c16 + _NCLS * t
        q = (_L - _NCLS) - _NCLS * t
        handles.append(
            pltpu.async_copy(w_ref.at[pl.ds(q, _L)],
                             out_hbm.at[i, :, pl.ds(h * _DH, _DH)],
                             sem))
    for hd in handles:
        hd.wait()


@jax.jit
def _rpe(table):
    e = jnp.concatenate([
        jnp.broadcast_to(table[0], (_FLANK, _D)),
        table,
        jnp.broadcast_to(table[_T_ROWS - 1], (_FLANK, _D)),
    ])
    pad = jnp.zeros((8, _E8_ROWS - _E_ROWS, _D), table.dtype)
    e8 = jnp.stack([
        jnp.concatenate([pad[0, :p_], e, pad[0, :_E8_ROWS - _E_ROWS - p_]])
        for p_ in range(8)
    ])

    mesh = plsc.VectorSubcoreMesh(core_axis_name="c", subcore_axis_name="s")
    return pl.kernel(
        _body,
        out_type=jax.ShapeDtypeStruct((_L, _L, _D), jnp.float32),
        mesh=mesh,
        scratch_types=[
            pltpu.VMEM((_W_ROWS, _DH), jnp.float32),
            pltpu.SemaphoreType.DMA,
        ],
        compiler_params=pltpu.CompilerParams(use_tc_tiling_on_sc=True),
    )(e8)


def kernel(seq_len, table):
    return _rpe(table)


# speedup vs baseline: 9.8356x; 1.2857x over previous
import jax
import jax.numpy as jnp
from jax import lax
from jax.experimental import pallas as pl
from jax.experimental.pallas import tpu as pltpu
from jax.experimental.pallas import tpu_sc as plsc

_MAX_DIST = 128
_D = 256
_L = 512
_T_ROWS = 2 * _MAX_DIST + 1

_NC = 2
_NS = 16

_DH = _D // 2
_E_ROWS = 2 * (_L - 1) + 1
_FLANK = _L - 1 - _MAX_DIST
_E8_ROWS = 1032
_W_ROWS = 1016
_NCLS = 16
_ROWS_PER_CLS = _L // _NCLS


def _body(e8_hbm, out_hbm, w_ref, sem):
    wid = lax.axis_index("s") * _NC + lax.axis_index("c")
    h = wid % 2
    c16 = wid // 2
    s0 = (_NCLS - 1) - c16
    p = lax.rem(c16 + 1, 8)
    S = pl.multiple_of(s0 + p, 8)

    pltpu.sync_copy(e8_hbm.at[p, pl.ds(S, _W_ROWS), pl.ds(h * _DH, _DH)],
                    w_ref)

    handles = []
    for t in range(_ROWS_PER_CLS):
        i = c16 + _NCLS * t
        q = (_L - _NCLS) - _NCLS * t
        handles.append(
            pltpu.async_copy(w_ref.at[pl.ds(q, _L)],
                             out_hbm.at[i, :, pl.ds(h * _DH, _DH)],
                             sem))
    for hd in handles:
        hd.wait()


@jax.jit
def _rpe(table):
    e_ext = jnp.concatenate([
        jnp.zeros((8, _D), table.dtype),
        jnp.broadcast_to(table[0], (_FLANK, _D)),
        table,
        jnp.broadcast_to(table[_T_ROWS - 1], (_FLANK, _D)),
        jnp.zeros((9, _D), table.dtype),
    ])
    e8 = jnp.stack([
        lax.slice(e_ext, (8 - p_, 0), (8 - p_ + _E8_ROWS, _D))
        for p_ in range(8)
    ])

    mesh = plsc.VectorSubcoreMesh(core_axis_name="c", subcore_axis_name="s")
    return pl.kernel(
        _body,
        out_type=jax.ShapeDtypeStruct((_L, _L, _D), jnp.float32),
        mesh=mesh,
        scratch_types=[
            pltpu.VMEM((_W_ROWS, _DH), jnp.float32),
            pltpu.SemaphoreType.DMA,
        ],
        compiler_params=pltpu.CompilerParams(use_tc_tiling_on_sc=True),
    )(e8)


def kernel(seq_len, table):
    return _rpe(table)


# speedup vs baseline: 10.0058x; 1.0173x over previous
import jax
import jax.numpy as jnp
from jax import lax
from jax.experimental import pallas as pl
from jax.experimental.pallas import tpu as pltpu
from jax.experimental.pallas import tpu_sc as plsc

_MAX_DIST = 128
_D = 256
_L = 512
_T_ROWS = 2 * _MAX_DIST + 1

_NC = 2
_NS = 16

_DH = _D // 2
_T8_ROWS = _T_ROWS + 7
_FLAT = 208
_W_ROWS = 1016
_NCLS = 16
_ROWS_PER_CLS = _L // _NCLS


def _body(t8_hbm, f0_hbm, f1_hbm, out_hbm, w_ref, sem):
    wid = lax.axis_index("s") * _NC + lax.axis_index("c")
    h = wid % 2
    c16 = wid // 2
    p = lax.rem(c16, 8)
    dst0 = jnp.where(c16 < 8, 368, 376)

    def al(x):
        return pl.multiple_of(x, 8)

    dh = pl.ds(h * _DH, _DH)

    fills = [
        pltpu.async_copy(t8_hbm.at[p, :, dh],
                         w_ref.at[pl.ds(al(dst0), _T8_ROWS)], sem),
        pltpu.async_copy(f0_hbm.at[pl.ds(0, _FLAT), dh],
                         w_ref.at[pl.ds(0, _FLAT)], sem),
        pltpu.async_copy(f0_hbm.at[pl.ds(0, 176), dh],
                         w_ref.at[pl.ds(al(dst0 - 176), 176)], sem),
        pltpu.async_copy(f1_hbm.at[pl.ds(0, _FLAT), dh],
                         w_ref.at[pl.ds(al(dst0 + _T8_ROWS), _FLAT)], sem),
        pltpu.async_copy(f1_hbm.at[pl.ds(0, 176), dh],
                         w_ref.at[pl.ds(_W_ROWS - 176, 176)], sem),
    ]
    for f in fills:
        f.wait()

    handles = []
    for t in range(_ROWS_PER_CLS):
        i = c16 + _NCLS * t
        q = (_L - _NCLS) - _NCLS * t
        handles.append(
            pltpu.async_copy(w_ref.at[pl.ds(q, _L)],
                             out_hbm.at[i, :, dh],
                             sem))
    for hd in handles:
        hd.wait()


@jax.jit
def _rpe(table):
    t0 = table[0]
    t256 = table[_T_ROWS - 1]
    base = jnp.concatenate([
        jnp.broadcast_to(t0, (7, _D)),
        table,
        jnp.broadcast_to(t256, (7, _D)),
    ])
    t8 = jnp.stack([
        lax.slice(base, (7 - p_, 0), (7 - p_ + _T8_ROWS, _D))
        for p_ in range(8)
    ])
    f0 = jnp.broadcast_to(t0, (_FLAT, _D))
    f1 = jnp.broadcast_to(t256, (_FLAT, _D))

    mesh = plsc.VectorSubcoreMesh(core_axis_name="c", subcore_axis_name="s")
    return pl.kernel(
        _body,
        out_type=jax.ShapeDtypeStruct((_L, _L, _D), jnp.float32),
        mesh=mesh,
        scratch_types=[
            pltpu.VMEM((_W_ROWS, _DH), jnp.float32),
            pltpu.SemaphoreType.DMA,
        ],
        compiler_params=pltpu.CompilerParams(use_tc_tiling_on_sc=True),
    )(t8, f0, f1)


def kernel(seq_len, table):
    return _rpe(table)
